# Initial kernel scaffold; baseline (speedup 1.0000x reference)
import jax
import jax.numpy as jnp
from jax.experimental import pallas as pl


def kernel(x, edge_index, batch, W1, a_src1, a_dst1, b1, W2, a_src2, a_dst2, b2):
    raise NotImplementedError("write your pallas kernel here")


# speedup vs baseline: 24.2955x; 24.2955x over previous
import functools
import jax
import jax.numpy as jnp
from jax import lax
from jax.experimental import pallas as pl
from jax.experimental.pallas import tpu as pltpu
import jax.experimental.pallas.tpu_sc as plsc

N = 10000
E = 320000
NG = 64
D_IN, D_HID, D_OUT = 128, 128, 16

NC, NS, L = 2, 16, 16
NW = NC * NS
NP = 10112
ROWS_PER_TILE = NP // NS
C1 = 144
C2 = 32
CH = 128
NCHUNK = 79
EP = NW * CH * NCHUNK
NEG = -1e30


def _prep1_body(x_ref, w1_ref, a2_ref, t_ref, av_ref, m_ref):
    h = jnp.dot(x_ref[...], w1_ref[...], preferred_element_type=jnp.float32)
    t_ref[:, 0:D_HID] = h
    colpat = jnp.where(
        lax.broadcasted_iota(jnp.int32, (NP, C1 - D_HID), 1) == 0, 1.0, 0.0)
    t_ref[:, D_HID:C1] = colpat
    av = lax.dot_general(a2_ref[...], h, (((1,), (1,)), ((), ())),
                         preferred_element_type=jnp.float32)
    col = lax.broadcasted_iota(jnp.int32, (2, NP), 1)
    av = jnp.where(col < N, av, NEG)
    av_ref[...] = av
    ms = jnp.max(av[0, :]) + jnp.max(av[1, :])
    m = jnp.where(ms >= 0.0, ms, 0.2 * ms)
    m_ref[...] = jnp.full((8, 128), m, jnp.float32)


def _mid_body(acc_ref, t1_ref, av1_ref, m1_ref, b1_ref, w2_ref, a2_ref,
              t2_ref, av2_ref, m2_ref):
    asrc = av1_ref[0, :]
    adst = av1_ref[1, :]
    m1 = m1_ref[0, 0]
    u = asrc + adst
    u = jnp.where(u >= 0.0, u, 0.2 * u)
    wself = jnp.exp(u - m1)
    num = (acc_ref[0, :, 0:D_HID] + acc_ref[1, :, 0:D_HID]
           + wself[:, None] * t1_ref[:, 0:D_HID])
    den = acc_ref[0, :, D_HID] + acc_ref[1, :, D_HID] + wself
    riota = lax.broadcasted_iota(jnp.int32, (NP,), 0)
    den = jnp.where(riota < N, den, 1.0)
    h = num / den[:, None] + b1_ref[...]
    h = jnp.maximum(h, 0.0)
    h2 = jnp.dot(h, w2_ref[...], preferred_element_type=jnp.float32)
    t2_ref[:, 0:D_OUT] = h2
    colpat = jnp.where(
        lax.broadcasted_iota(jnp.int32, (NP, C2 - D_OUT), 1) == 0, 1.0, 0.0)
    t2_ref[:, D_OUT:C2] = colpat
    av = lax.dot_general(a2_ref[...], h2, (((1,), (1,)), ((), ())),
                         preferred_element_type=jnp.float32)
    col = lax.broadcasted_iota(jnp.int32, (2, NP), 1)
    av = jnp.where(col < N, av, NEG)
    av2_ref[...] = av
    ms = jnp.max(av[0, :]) + jnp.max(av[1, :])
    m = jnp.where(ms >= 0.0, ms, 0.2 * ms)
    m2_ref[...] = jnp.full((8, 128), m, jnp.float32)


def _final_body(acc_ref, t2_ref, av2_ref, m2_ref, b2_ref, batch_ref, out_ref):
    asrc = av2_ref[0, 0:N]
    adst = av2_ref[1, 0:N]
    m2 = m2_ref[0, 0]
    u = asrc + adst
    u = jnp.where(u >= 0.0, u, 0.2 * u)
    wself = jnp.exp(u - m2)
    num = (acc_ref[0, 0:N, 0:D_OUT] + acc_ref[1, 0:N, 0:D_OUT]
           + wself[:, None] * t2_ref[0:N, 0:D_OUT])
    den = acc_ref[0, 0:N, D_OUT] + acc_ref[1, 0:N, D_OUT] + wself
    h2 = num / den[:, None] + b2_ref[...]
    b = batch_ref[0, :]
    gid = lax.broadcasted_iota(jnp.int32, (NG, N), 0)
    mask = jnp.where(gid == b[None, :], 1.0, 0.0)
    counts = jnp.sum(mask, axis=1)
    sums = jnp.dot(mask, h2, preferred_element_type=jnp.float32)
    pooled = sums / jnp.maximum(counts, 1.0)[:, None]
    zmax = jnp.max(pooled, axis=1, keepdims=True)
    z = pooled - zmax
    out_ref[...] = z - jnp.log(jnp.sum(jnp.exp(z), axis=1, keepdims=True))


_prep1 = pl.pallas_call(
    _prep1_body,
    out_shape=[jax.ShapeDtypeStruct((NP, C1), jnp.float32),
               jax.ShapeDtypeStruct((2, NP), jnp.float32),
               jax.ShapeDtypeStruct((8, 128), jnp.float32)],
)

_mid = pl.pallas_call(
    _mid_body,
    out_shape=[jax.ShapeDtypeStruct((NP, C2), jnp.float32),
               jax.ShapeDtypeStruct((2, NP), jnp.float32),
               jax.ShapeDtypeStruct((8, 128), jnp.float32)],
)

_final = pl.pallas_call(
    _final_body,
    out_shape=jax.ShapeDtypeStruct((NG, D_OUT), jnp.float32),
)


def _make_edge_kernel(cols):
    grp = cols // L
    mesh = plsc.VectorSubcoreMesh(core_axis_name="c", subcore_axis_name="s",
                                  num_cores=NC, num_subcores=NS)

    @functools.partial(
        pl.kernel,
        out_type=jax.ShapeDtypeStruct((NC, NP, cols), jnp.float32),
        mesh=mesh,
        compiler_params=pltpu.CompilerParams(needs_layout_passes=False,
                                             use_tc_tiling_on_sc=False),
        scratch_types=[
            pltpu.VMEM((NP,), jnp.float32),
            pltpu.VMEM((NP,), jnp.float32),
            pltpu.VMEM((L,), jnp.float32),
            pltpu.VMEM((CH,), jnp.int32),
            pltpu.VMEM((CH,), jnp.int32),
            pltpu.VMEM((CH,), jnp.float32),
            pltpu.VMEM((CH, cols), jnp.float32),
            pltpu.VMEM_SHARED((NP, cols), jnp.float32),
            pltpu.SemaphoreType.DMA,
        ],
    )
    def edge_kernel(src_hbm, dst_hbm, tab_hbm, av_hbm, m_hbm, zeros_hbm,
                    out_hbm, asrc_t, adst_t, m_v, src_v, dst_v, w_v, rows_v,
                    acc, sem):
        cid = lax.axis_index("c")
        sid = lax.axis_index("s")
        wid = cid * NS + sid
        pltpu.sync_copy(av_hbm.at[0], asrc_t)
        pltpu.sync_copy(av_hbm.at[1], adst_t)
        pltpu.sync_copy(m_hbm, m_v)
        r0 = sid * ROWS_PER_TILE
        pltpu.sync_copy(zeros_hbm.at[pl.ds(r0, ROWS_PER_TILE)],
                        acc.at[pl.ds(r0, ROWS_PER_TILE)])
        plsc.subcore_barrier()
        mvec = m_v[...]
        base_t = wid * (CH * NCHUNK)

        def chunk_body(k, carry):
            base = base_t + k * CH
            pltpu.sync_copy(src_hbm.at[pl.ds(base, CH)], src_v)
            pltpu.sync_copy(dst_hbm.at[pl.ds(base, CH)], dst_v)
            cp = pltpu.async_copy(tab_hbm.at[src_v], rows_v, sem)

            def w_body(i, c2):
                si = src_v[pl.ds(i * L, L)]
                di = dst_v[pl.ds(i * L, L)]
                a = plsc.load_gather(asrc_t, [si]) + plsc.load_gather(adst_t, [di])
                a = jnp.where(a >= 0.0, a, 0.2 * a)
                w_v[pl.ds(i * L, L)] = jnp.exp(a - mvec)
                return c2

            lax.fori_loop(0, CH // L, w_body, 0)
            cp.wait()

            def s_body(e, c2):
                we = plsc.load_gather(w_v, [jnp.full((L,), e, jnp.int32)])
                for j in range(grp):
                    rows_v[e, pl.ds(j * L, L)] = rows_v[e, pl.ds(j * L, L)] * we
                return c2

            lax.fori_loop(0, CH, s_body, 0)
            pltpu.sync_copy(rows_v, acc.at[dst_v], add=True)
            return carry

        lax.fori_loop(0, NCHUNK, chunk_body, 0)
        plsc.subcore_barrier()
        pltpu.sync_copy(acc.at[pl.ds(r0, ROWS_PER_TILE)],
                        out_hbm.at[cid, pl.ds(r0, ROWS_PER_TILE)])

    return edge_kernel


_edge1 = _make_edge_kernel(C1)
_edge2 = _make_edge_kernel(C2)


@jax.jit
def kernel(x, edge_index, batch, W1, a_src1, a_dst1, b1, W2, a_src2, a_dst2, b2):
    x = x.astype(jnp.float32)
    src = edge_index[0].astype(jnp.int32)
    dst = edge_index[1].astype(jnp.int32)
    batch = batch.astype(jnp.int32)

    x_pad = jnp.zeros((NP, D_IN), jnp.float32).at[0:N].set(x)
    pad = jnp.full((EP - E,), N, jnp.int32)
    src_pad = jnp.concatenate([src, pad])
    dst_pad = jnp.concatenate([dst, pad])
    a2_1 = jnp.stack([a_src1, a_dst1])
    a2_2 = jnp.stack([a_src2, a_dst2])
    zeros1 = jnp.zeros((NP, C1), jnp.float32)
    zeros2 = jnp.zeros((NP, C2), jnp.float32)

    t1, av1, m1 = _prep1(x_pad, W1, a2_1)
    acc1 = _edge1(src_pad, dst_pad, t1, av1, m1[0, 0:L], zeros1)
    t2, av2, m2 = _mid(acc1, t1, av1, m1, b1, W2, a2_2)
    acc2 = _edge2(src_pad, dst_pad, t2, av2, m2[0, 0:L], zeros2)
    out = _final(acc2, t2, av2, m2, b2, batch.reshape(1, N))
    return out


# speedup vs baseline: 33.0723x; 1.3613x over previous
import functools
import jax
import jax.numpy as jnp
from jax import lax
from jax.experimental import pallas as pl
from jax.experimental.pallas import tpu as pltpu
import jax.experimental.pallas.tpu_sc as plsc

N = 10000
E = 320000
NG = 64
D_IN, D_HID, D_OUT = 128, 128, 16

NC, NS, L = 2, 16, 16
NW = NC * NS
NP = 10112
ROWS_PER_TILE = NP // NS
C1 = 128
C2 = 16
CH = 80
NCHUNK = 128
EP = NW * CH * NCHUNK
NEG = -1e30


def _prep1_body(x_ref, w1_ref, a2_ref, t_ref, av_ref, m_ref):
    h = jnp.dot(x_ref[...], w1_ref[...], preferred_element_type=jnp.float32)
    t_ref[...] = h
    av = lax.dot_general(a2_ref[...], h, (((1,), (1,)), ((), ())),
                         preferred_element_type=jnp.float32)
    col = lax.broadcasted_iota(jnp.int32, (2, NP), 1)
    av = jnp.where(col < N, av, NEG)
    av_ref[...] = av
    ms = jnp.max(av[0, :]) + jnp.max(av[1, :])
    m = jnp.where(ms >= 0.0, ms, 0.2 * ms)
    m_ref[...] = jnp.full((8, 128), m, jnp.float32)


def _mid_body(acc_ref, dacc_ref, t1_ref, av1_ref, m1_ref, b1_ref, w2_ref,
              a2_ref, t2_ref, av2_ref, m2_ref):
    asrc = av1_ref[0, :]
    adst = av1_ref[1, :]
    m1 = m1_ref[0, 0]
    u = asrc + adst
    u = jnp.where(u >= 0.0, u, 0.2 * u)
    wself = jnp.exp(u - m1)
    num = (acc_ref[0, :, :] + acc_ref[1, :, :]
           + wself[:, None] * t1_ref[...])
    den = dacc_ref[0, :] + dacc_ref[1, :] + wself
    riota = lax.broadcasted_iota(jnp.int32, (NP,), 0)
    den = jnp.where(riota < N, den, 1.0)
    h = num / den[:, None] + b1_ref[...]
    h = jnp.maximum(h, 0.0)
    h2 = jnp.dot(h, w2_ref[...], preferred_element_type=jnp.float32)
    t2_ref[...] = h2
    av = lax.dot_general(a2_ref[...], h2, (((1,), (1,)), ((), ())),
                         preferred_element_type=jnp.float32)
    col = lax.broadcasted_iota(jnp.int32, (2, NP), 1)
    av = jnp.where(col < N, av, NEG)
    av2_ref[...] = av
    ms = jnp.max(av[0, :]) + jnp.max(av[1, :])
    m = jnp.where(ms >= 0.0, ms, 0.2 * ms)
    m2_ref[...] = jnp.full((8, 128), m, jnp.float32)


def _final_body(acc_ref, dacc_ref, t2_ref, av2_ref, m2_ref, b2_ref, batch_ref,
                out_ref):
    asrc = av2_ref[0, 0:N]
    adst = av2_ref[1, 0:N]
    m2 = m2_ref[0, 0]
    u = asrc + adst
    u = jnp.where(u >= 0.0, u, 0.2 * u)
    wself = jnp.exp(u - m2)
    num = (acc_ref[0, 0:N, 0:D_OUT] + acc_ref[1, 0:N, 0:D_OUT]
           + wself[:, None] * t2_ref[0:N, 0:D_OUT])
    den = dacc_ref[0, 0:N] + dacc_ref[1, 0:N] + wself
    h2 = num / den[:, None] + b2_ref[...]
    b = batch_ref[0, :]
    gid = lax.broadcasted_iota(jnp.int32, (NG, N), 0)
    mask = jnp.where(gid == b[None, :], 1.0, 0.0)
    counts = jnp.sum(mask, axis=1)
    sums = jnp.dot(mask, h2, preferred_element_type=jnp.float32)
    pooled = sums / jnp.maximum(counts, 1.0)[:, None]
    zmax = jnp.max(pooled, axis=1, keepdims=True)
    z = pooled - zmax
    out_ref[...] = z - jnp.log(jnp.sum(jnp.exp(z), axis=1, keepdims=True))


_prep1 = pl.pallas_call(
    _prep1_body,
    out_shape=[jax.ShapeDtypeStruct((NP, C1), jnp.float32),
               jax.ShapeDtypeStruct((2, NP), jnp.float32),
               jax.ShapeDtypeStruct((8, 128), jnp.float32)],
)

_mid = pl.pallas_call(
    _mid_body,
    out_shape=[jax.ShapeDtypeStruct((NP, C2), jnp.float32),
               jax.ShapeDtypeStruct((2, NP), jnp.float32),
               jax.ShapeDtypeStruct((8, 128), jnp.float32)],
)

_final = pl.pallas_call(
    _final_body,
    out_shape=jax.ShapeDtypeStruct((NG, D_OUT), jnp.float32),
)


def _make_edge_kernel(cols):
    grp = cols // L
    mesh = plsc.VectorSubcoreMesh(core_axis_name="c", subcore_axis_name="s",
                                  num_cores=NC, num_subcores=NS)

    @functools.partial(
        pl.kernel,
        out_type=[jax.ShapeDtypeStruct((NC, NP, cols), jnp.float32),
                  jax.ShapeDtypeStruct((NC, NP), jnp.float32)],
        mesh=mesh,
        compiler_params=pltpu.CompilerParams(needs_layout_passes=False,
                                             use_tc_tiling_on_sc=False),
        scratch_types=[
            pltpu.VMEM((L,), jnp.float32),
            pltpu.VMEM((NCHUNK, CH), jnp.int32),
            pltpu.VMEM((NCHUNK, CH), jnp.int32),
            [pltpu.VMEM((CH,), jnp.float32)] * 2,
            [pltpu.VMEM((CH,), jnp.float32)] * 2,
            [pltpu.VMEM((CH,), jnp.float32)] * 2,
            [pltpu.VMEM((CH, cols), jnp.float32)] * 2,
            pltpu.VMEM_SHARED((NP, cols), jnp.float32),
            pltpu.VMEM_SHARED((NP,), jnp.float32),
            [pltpu.SemaphoreType.DMA] * 2,
            [pltpu.SemaphoreType.DMA] * 2,
            [pltpu.SemaphoreType.DMA] * 2,
            [pltpu.SemaphoreType.DMA] * 2,
            [pltpu.SemaphoreType.DMA] * 2,
        ],
    )
    def edge_kernel(src_hbm, dst_hbm, tab_hbm, asrc_hbm, adst_hbm, m_hbm,
                    zeros_hbm, zerosd_hbm, out_hbm, den_hbm, m_v, src_t,
                    dst_t, avs_v, avd_v, w_v, rows_v, acc, dacc, gsem, asem,
                    dsem, ssem, wsem):
        cid = lax.axis_index("c")
        sid = lax.axis_index("s")
        wid = cid * NS + sid
        pltpu.sync_copy(m_hbm, m_v)
        r0 = sid * ROWS_PER_TILE
        pltpu.sync_copy(zeros_hbm.at[pl.ds(r0, ROWS_PER_TILE)],
                        acc.at[pl.ds(r0, ROWS_PER_TILE)])
        pltpu.sync_copy(zerosd_hbm.at[pl.ds(r0, ROWS_PER_TILE)],
                        dacc.at[pl.ds(r0, ROWS_PER_TILE)])
        pltpu.sync_copy(src_hbm.at[wid], src_t)
        pltpu.sync_copy(dst_hbm.at[wid], dst_t)
        plsc.subcore_barrier()
        mvec = m_v[...]

        pltpu.async_copy(tab_hbm.at[src_t.at[0]], rows_v[0], gsem[0])
        pltpu.async_copy(asrc_hbm.at[src_t.at[0]], avs_v[0], asem[0])
        pltpu.async_copy(adst_hbm.at[dst_t.at[0]], avd_v[0], dsem[0])

        def pair_body(p, carry):
            for b in range(2):
                ob = 1 - b
                k = 2 * p + b

                pltpu.make_async_copy(asrc_hbm.at[src_t.at[0]], avs_v[b],
                                      asem[b]).wait()
                pltpu.make_async_copy(adst_hbm.at[dst_t.at[0]], avd_v[b],
                                      dsem[b]).wait()

                def w_body(i, c2, b=b):
                    a = (avs_v[b][pl.ds(i * L, L)]
                         + avd_v[b][pl.ds(i * L, L)])
                    a = jnp.where(a >= 0.0, a, 0.2 * a)
                    w_v[b][pl.ds(i * L, L)] = jnp.exp(a - mvec)
                    return c2

                lax.fori_loop(0, CH // L, w_body, 0)

                def wait_scatter(ob=ob):
                    pltpu.make_async_copy(rows_v[ob], acc.at[dst_t.at[0]],
                                          ssem[ob]).wait()
                    pltpu.make_async_copy(w_v[ob], dacc.at[dst_t.at[0]],
                                          wsem[ob]).wait()
                if b == 0:
                    pl.when(p >= 1)(wait_scatter)
                else:
                    wait_scatter()

                nk = jnp.minimum(k + 1, NCHUNK - 1)
                pltpu.async_copy(tab_hbm.at[src_t.at[nk]], rows_v[ob],
                                 gsem[ob])
                pltpu.async_copy(asrc_hbm.at[src_t.at[nk]], avs_v[ob],
                                 asem[ob])
                pltpu.async_copy(adst_hbm.at[dst_t.at[nk]], avd_v[ob],
                                 dsem[ob])

                pltpu.make_async_copy(tab_hbm.at[src_t.at[0]], rows_v[b],
                                      gsem[b]).wait()

                def s_body(e, c2, b=b):
                    we = plsc.load_gather(w_v[b],
                                          [jnp.full((L,), e, jnp.int32)])
                    for j in range(grp):
                        rows_v[b][e, pl.ds(j * L, L)] = (
                            rows_v[b][e, pl.ds(j * L, L)] * we)
                    return c2

                lax.fori_loop(0, CH, s_body, 0)
                pltpu.async_copy(rows_v[b], acc.at[dst_t.at[k]], ssem[b],
                                 add=True)
                pltpu.async_copy(w_v[b], dacc.at[dst_t.at[k]], wsem[b],
                                 add=True)
            return carry

        lax.fori_loop(0, NCHUNK // 2, pair_body, 0)
        pltpu.make_async_copy(tab_hbm.at[src_t.at[0]], rows_v[0],
                              gsem[0]).wait()
        pltpu.make_async_copy(asrc_hbm.at[src_t.at[0]], avs_v[0],
                              asem[0]).wait()
        pltpu.make_async_copy(adst_hbm.at[dst_t.at[0]], avd_v[0],
                              dsem[0]).wait()
        pltpu.make_async_copy(rows_v[1], acc.at[dst_t.at[0]], ssem[1]).wait()
        pltpu.make_async_copy(w_v[1], dacc.at[dst_t.at[0]], wsem[1]).wait()
        plsc.subcore_barrier()
        pltpu.sync_copy(acc.at[pl.ds(r0, ROWS_PER_TILE)],
                        out_hbm.at[cid, pl.ds(r0, ROWS_PER_TILE)])
        pltpu.sync_copy(dacc.at[pl.ds(r0, ROWS_PER_TILE)],
                        den_hbm.at[cid, pl.ds(r0, ROWS_PER_TILE)])

    return edge_kernel


_edge1 = _make_edge_kernel(C1)
_edge2 = _make_edge_kernel(C2)


@jax.jit
def kernel(x, edge_index, batch, W1, a_src1, a_dst1, b1, W2, a_src2, a_dst2, b2):
    x = x.astype(jnp.float32)
    src = edge_index[0].astype(jnp.int32)
    dst = edge_index[1].astype(jnp.int32)
    batch = batch.astype(jnp.int32)

    x_pad = jnp.zeros((NP, D_IN), jnp.float32).at[0:N].set(x)
    pad = jnp.full((EP - E,), N, jnp.int32)
    src_pad = jnp.concatenate([src, pad]).reshape(NW, NCHUNK, CH)
    dst_pad = jnp.concatenate([dst, pad]).reshape(NW, NCHUNK, CH)
    a2_1 = jnp.stack([a_src1, a_dst1])
    a2_2 = jnp.stack([a_src2, a_dst2])
    zeros1 = jnp.zeros((NP, C1), jnp.float32)
    zeros2 = jnp.zeros((NP, C2), jnp.float32)
    zerosd = jnp.zeros((NP,), jnp.float32)

    t1, av1, m1 = _prep1(x_pad, W1, a2_1)
    acc1, den1 = _edge1(src_pad, dst_pad, t1, av1[0], av1[1], m1[0, 0:L],
                        zeros1, zerosd)
    t2, av2, m2 = _mid(acc1, den1, t1, av1, m1, b1, W2, a2_2)
    acc2, den2 = _edge2(src_pad, dst_pad, t2, av2[0], av2[1], m2[0, 0:L],
                        zeros2, zerosd)
    out = _final(acc2, den2, t2, av2, m2, b2, batch.reshape(1, N))
    return out


# speedup vs baseline: 33.4258x; 1.0107x over previous
import functools
import jax
import jax.numpy as jnp
from jax import lax
from jax.experimental import pallas as pl
from jax.experimental.pallas import tpu as pltpu
import jax.experimental.pallas.tpu_sc as plsc

N = 10000
E = 320000
NG = 64
D_IN, D_HID, D_OUT = 128, 128, 16

NC, NS, L = 2, 16, 16
NW = NC * NS
NP = 10112
ROWS_PER_TILE = NP // NS
C1 = 128
C2 = 16
CH = 80
NCHUNK = 128
EP = NW * CH * NCHUNK
NEG = -1e30


def _prep1_body(x_ref, w1_ref, a2_ref, t_ref, av_ref, m_ref):
    h = jnp.dot(x_ref[...], w1_ref[...], preferred_element_type=jnp.float32)
    t_ref[...] = h
    av = lax.dot_general(a2_ref[...], h, (((1,), (1,)), ((), ())),
                         preferred_element_type=jnp.float32)
    col = lax.broadcasted_iota(jnp.int32, (2, NP), 1)
    av = jnp.where(col < N, av, NEG)
    av_ref[...] = av
    ms = jnp.max(av[0, :]) + jnp.max(av[1, :])
    m = jnp.where(ms >= 0.0, ms, 0.2 * ms)
    m_ref[...] = jnp.full((8, 128), m, jnp.float32)


def _mid_body(acc_ref, dacc_ref, t1_ref, av1_ref, m1_ref, b1_ref, w2_ref,
              a2_ref, t2_ref, av2_ref, m2_ref):
    asrc = av1_ref[0, :]
    adst = av1_ref[1, :]
    m1 = m1_ref[0, 0]
    u = asrc + adst
    u = jnp.where(u >= 0.0, u, 0.2 * u)
    wself = jnp.exp(u - m1)
    num = (acc_ref[0, :, :] + acc_ref[1, :, :]
           + wself[:, None] * t1_ref[...])
    den = dacc_ref[0, :] + dacc_ref[1, :] + wself
    riota = lax.broadcasted_iota(jnp.int32, (NP,), 0)
    den = jnp.where(riota < N, den, 1.0)
    h = num / den[:, None] + b1_ref[...]
    h = jnp.maximum(h, 0.0)
    h2 = jnp.dot(h, w2_ref[...], preferred_element_type=jnp.float32)
    t2_ref[...] = h2
    av = lax.dot_general(a2_ref[...], h2, (((1,), (1,)), ((), ())),
                         preferred_element_type=jnp.float32)
    col = lax.broadcasted_iota(jnp.int32, (2, NP), 1)
    av = jnp.where(col < N, av, NEG)
    av2_ref[...] = av
    ms = jnp.max(av[0, :]) + jnp.max(av[1, :])
    m = jnp.where(ms >= 0.0, ms, 0.2 * ms)
    m2_ref[...] = jnp.full((8, 128), m, jnp.float32)


def _final_body(acc_ref, dacc_ref, t2_ref, av2_ref, m2_ref, b2_ref, batch_ref,
                out_ref):
    asrc = av2_ref[0, 0:N]
    adst = av2_ref[1, 0:N]
    m2 = m2_ref[0, 0]
    u = asrc + adst
    u = jnp.where(u >= 0.0, u, 0.2 * u)
    wself = jnp.exp(u - m2)
    num = (acc_ref[0, 0:N, 0:D_OUT] + acc_ref[1, 0:N, 0:D_OUT]
           + wself[:, None] * t2_ref[0:N, 0:D_OUT])
    den = dacc_ref[0, 0:N] + dacc_ref[1, 0:N] + wself
    h2 = num / den[:, None] + b2_ref[...]
    b = batch_ref[0, :]
    gid = lax.broadcasted_iota(jnp.int32, (NG, N), 0)
    mask = jnp.where(gid == b[None, :], 1.0, 0.0)
    counts = jnp.sum(mask, axis=1)
    sums = jnp.dot(mask, h2, preferred_element_type=jnp.float32)
    pooled = sums / jnp.maximum(counts, 1.0)[:, None]
    zmax = jnp.max(pooled, axis=1, keepdims=True)
    z = pooled - zmax
    out_ref[...] = z - jnp.log(jnp.sum(jnp.exp(z), axis=1, keepdims=True))


_prep1 = pl.pallas_call(
    _prep1_body,
    out_shape=[jax.ShapeDtypeStruct((NP, C1), jnp.float32),
               jax.ShapeDtypeStruct((2, NP), jnp.float32),
               jax.ShapeDtypeStruct((8, 128), jnp.float32)],
)

_mid = pl.pallas_call(
    _mid_body,
    out_shape=[jax.ShapeDtypeStruct((NP, C2), jnp.float32),
               jax.ShapeDtypeStruct((2, NP), jnp.float32),
               jax.ShapeDtypeStruct((8, 128), jnp.float32)],
)

_final = pl.pallas_call(
    _final_body,
    out_shape=jax.ShapeDtypeStruct((NG, D_OUT), jnp.float32),
)


def _make_edge_kernel(cols):
    grp = cols // L
    mesh = plsc.VectorSubcoreMesh(core_axis_name="c", subcore_axis_name="s",
                                  num_cores=NC, num_subcores=NS)

    @functools.partial(
        pl.kernel,
        out_type=[jax.ShapeDtypeStruct((NC, NP, cols), jnp.float32),
                  jax.ShapeDtypeStruct((NC, NP), jnp.float32)],
        mesh=mesh,
        compiler_params=pltpu.CompilerParams(needs_layout_passes=False,
                                             use_tc_tiling_on_sc=False),
        scratch_types=[
            pltpu.VMEM((L,), jnp.float32),
            pltpu.VMEM((NCHUNK, CH), jnp.int32),
            pltpu.VMEM((NCHUNK, CH), jnp.int32),
            [pltpu.VMEM((CH,), jnp.float32)] * 2,
            [pltpu.VMEM((CH,), jnp.float32)] * 2,
            [pltpu.VMEM((CH,), jnp.float32)] * 2,
            [pltpu.VMEM((CH, cols), jnp.float32)] * 2,
            pltpu.VMEM_SHARED((NP, cols), jnp.float32),
            pltpu.VMEM_SHARED((NP,), jnp.float32),
            [pltpu.SemaphoreType.DMA] * 2,
            [pltpu.SemaphoreType.DMA] * 2,
            [pltpu.SemaphoreType.DMA] * 2,
            [pltpu.SemaphoreType.DMA] * 2,
            [pltpu.SemaphoreType.DMA] * 2,
        ],
    )
    def edge_kernel(src_hbm, dst_hbm, tab_hbm, asrc_hbm, adst_hbm, m_hbm,
                    zeros_hbm, zerosd_hbm, out_hbm, den_hbm, m_v, src_t,
                    dst_t, avs_v, avd_v, w_v, rows_v, acc, dacc, gsem, asem,
                    dsem, ssem, wsem):
        cid = lax.axis_index("c")
        sid = lax.axis_index("s")
        wid = cid * NS + sid
        pltpu.sync_copy(m_hbm, m_v)
        r0 = sid * ROWS_PER_TILE
        pltpu.sync_copy(zeros_hbm.at[pl.ds(r0, ROWS_PER_TILE)],
                        acc.at[pl.ds(r0, ROWS_PER_TILE)])
        pltpu.sync_copy(zerosd_hbm.at[pl.ds(r0, ROWS_PER_TILE)],
                        dacc.at[pl.ds(r0, ROWS_PER_TILE)])
        pltpu.sync_copy(src_hbm.at[wid], src_t)
        pltpu.sync_copy(dst_hbm.at[wid], dst_t)
        plsc.subcore_barrier()
        mvec = m_v[...]

        pltpu.async_copy(tab_hbm.at[src_t.at[0]], rows_v[0], gsem[0])
        pltpu.async_copy(asrc_hbm.at[src_t.at[0]], avs_v[0], asem[0])
        pltpu.async_copy(adst_hbm.at[dst_t.at[0]], avd_v[0], dsem[0])

        def pair_body(p, carry):
            for b in range(2):
                ob = 1 - b
                k = 2 * p + b

                pltpu.make_async_copy(asrc_hbm.at[src_t.at[0]], avs_v[b],
                                      asem[b]).wait()
                pltpu.make_async_copy(adst_hbm.at[dst_t.at[0]], avd_v[b],
                                      dsem[b]).wait()

                def w_body(i, c2, b=b):
                    a = (avs_v[b][pl.ds(i * L, L)]
                         + avd_v[b][pl.ds(i * L, L)])
                    a = jnp.where(a >= 0.0, a, 0.2 * a)
                    w_v[b][pl.ds(i * L, L)] = jnp.exp(a - mvec)
                    return c2

                lax.fori_loop(0, CH // L, w_body, 0)

                def wait_scatter(ob=ob):
                    pltpu.make_async_copy(rows_v[ob], acc.at[dst_t.at[0]],
                                          ssem[ob]).wait()
                    pltpu.make_async_copy(w_v[ob], dacc.at[dst_t.at[0]],
                                          wsem[ob]).wait()
                if b == 0:
                    pl.when(p >= 1)(wait_scatter)
                else:
                    wait_scatter()

                nk = jnp.minimum(k + 1, NCHUNK - 1)
                pltpu.async_copy(tab_hbm.at[src_t.at[nk]], rows_v[ob],
                                 gsem[ob])
                pltpu.async_copy(asrc_hbm.at[src_t.at[nk]], avs_v[ob],
                                 asem[ob])
                pltpu.async_copy(adst_hbm.at[dst_t.at[nk]], avd_v[ob],
                                 dsem[ob])

                pltpu.make_async_copy(tab_hbm.at[src_t.at[0]], rows_v[b],
                                      gsem[b]).wait()

                @plsc.parallel_loop(0, CH, unroll=4)
                def _(e, b=b):
                    we = plsc.load_gather(w_v[b],
                                          [jnp.full((L,), e, jnp.int32)])
                    for j in range(grp):
                        rows_v[b][e, pl.ds(j * L, L)] = (
                            rows_v[b][e, pl.ds(j * L, L)] * we)
                pltpu.async_copy(rows_v[b], acc.at[dst_t.at[k]], ssem[b],
                                 add=True)
                pltpu.async_copy(w_v[b], dacc.at[dst_t.at[k]], wsem[b],
                                 add=True)
            return carry

        lax.fori_loop(0, NCHUNK // 2, pair_body, 0)
        pltpu.make_async_copy(tab_hbm.at[src_t.at[0]], rows_v[0],
                              gsem[0]).wait()
        pltpu.make_async_copy(asrc_hbm.at[src_t.at[0]], avs_v[0],
                              asem[0]).wait()
        pltpu.make_async_copy(adst_hbm.at[dst_t.at[0]], avd_v[0],
                              dsem[0]).wait()
        pltpu.make_async_copy(rows_v[1], acc.at[dst_t.at[0]], ssem[1]).wait()
        pltpu.make_async_copy(w_v[1], dacc.at[dst_t.at[0]], wsem[1]).wait()
        plsc.subcore_barrier()
        pltpu.sync_copy(acc.at[pl.ds(r0, ROWS_PER_TILE)],
                        out_hbm.at[cid, pl.ds(r0, ROWS_PER_TILE)])
        pltpu.sync_copy(dacc.at[pl.ds(r0, ROWS_PER_TILE)],
                        den_hbm.at[cid, pl.ds(r0, ROWS_PER_TILE)])

    return edge_kernel


_edge1 = _make_edge_kernel(C1)
_edge2 = _make_edge_kernel(C2)


@jax.jit
def kernel(x, edge_index, batch, W1, a_src1, a_dst1, b1, W2, a_src2, a_dst2, b2):
    x = x.astype(jnp.float32)
    src = edge_index[0].astype(jnp.int32)
    dst = edge_index[1].astype(jnp.int32)
    batch = batch.astype(jnp.int32)

    x_pad = jnp.zeros((NP, D_IN), jnp.float32).at[0:N].set(x)
    pad = jnp.full((EP - E,), N, jnp.int32)
    src_pad = jnp.concatenate([src, pad]).reshape(NW, NCHUNK, CH)
    dst_pad = jnp.concatenate([dst, pad]).reshape(NW, NCHUNK, CH)
    a2_1 = jnp.stack([a_src1, a_dst1])
    a2_2 = jnp.stack([a_src2, a_dst2])
    zeros1 = jnp.zeros((NP, C1), jnp.float32)
    zeros2 = jnp.zeros((NP, C2), jnp.float32)
    zerosd = jnp.zeros((NP,), jnp.float32)

    t1, av1, m1 = _prep1(x_pad, W1, a2_1)
    acc1, den1 = _edge1(src_pad, dst_pad, t1, av1[0], av1[1], m1[0, 0:L],
                        zeros1, zerosd)
    t2, av2, m2 = _mid(acc1, den1, t1, av1, m1, b1, W2, a2_2)
    acc2, den2 = _edge2(src_pad, dst_pad, t2, av2[0], av2[1], m2[0, 0:L],
                        zeros2, zerosd)
    out = _final(acc2, den2, t2, av2, m2, b2, batch.reshape(1, N))
    return out
